# baseline (speedup 1.0000x reference)
import jax
import jax.numpy as jnp
from jax import lax
from jax.experimental import pallas as pl
from jax.experimental.pallas import tpu as pltpu
from jax.experimental.pallas import tpu_sc as plsc

B = 4096
SEQ = 50
DW = 300
VR = 1000001
N_OUT = 4
LANES = 16
PD = 128
NW = 32
RPW = B // NW
G = 4
NCH = RPW // G
GIDX = G * SEQ
SPLIT = 104
IDXW = RPW * SEQ
BM = 8192


def _lgather(ref, idx):
    return plsc.load_gather(ref, [idx])


KB = 128
KSTEPS = pl.cdiv(DW, KB)


def _proj_body(w_ref, f_ref, out_ref):
    k = pl.program_id(1)
    live = jnp.where(k == KSTEPS - 1, DW - KB * (KSTEPS - 1), KB)
    col = lax.broadcasted_iota(jnp.int32, (PD, KB), 1)
    f = jnp.where(col < live, f_ref[...], 0.0).astype(jnp.bfloat16)
    partial = lax.dot_general(
        w_ref[...].astype(jnp.bfloat16), f,
        (((1,), (1,)), ((), ())),
        preferred_element_type=jnp.float32)

    @pl.when(k == 0)
    def _():
        out_ref[...] = partial

    @pl.when(k > 0)
    def _():
        out_ref[...] += partial


_proj = pl.pallas_call(
    _proj_body,
    grid=(pl.cdiv(VR, BM), KSTEPS),
    in_specs=[
        pl.BlockSpec((BM, KB), lambda i, k: (i, k)),
        pl.BlockSpec((PD, KB), lambda i, k: (0, k)),
    ],
    out_specs=pl.BlockSpec((BM, PD), lambda i, k: (i, 0)),
    out_shape=jax.ShapeDtypeStruct((VR, PD), jnp.float32),
)


def _pool_body(x_hbm, p_hbm, bias_hbm, out_hbm,
               idx_v, buf0, buf1, bias_v, outst_v, tsc_v,
               idxa0, idxb0, idxa1, idxb1, sem0, sem1):
    cid = lax.axis_index("c")
    sid = lax.axis_index("s")
    wid = sid * 2 + cid

    pltpu.sync_copy(x_hbm.at[pl.ds(pl.multiple_of(wid * IDXW, 8), IDXW)], idx_v)
    pltpu.sync_copy(bias_hbm, bias_v)

    bufs = (buf0, buf1)
    sems = (sem0, sem1)
    idxas = (idxa0, idxa1)
    idxbs = (idxb0, idxb1)

    def _gather_descs(b):
        d0 = pltpu.make_async_copy(
            p_hbm.at[idxas[b]], bufs[b].at[pl.ds(0, SPLIT)], sems[b])
        d1 = pltpu.make_async_copy(
            p_hbm.at[idxbs[b]], bufs[b].at[pl.ds(SPLIT, GIDX - SPLIT)],
            sems[b])
        return d0, d1

    def _start(g, b):
        off = g * GIDX
        for k in range(SPLIT // LANES):
            idxas[b][pl.ds(k * LANES, LANES)] = \
                idx_v[pl.ds(off + k * LANES, LANES)]
        idxas[b][pl.ds(SPLIT - LANES, LANES)] = \
            idx_v[pl.ds(off + SPLIT - LANES, LANES)]
        for k in range((GIDX - SPLIT) // LANES):
            idxbs[b][pl.ds(k * LANES, LANES)] = \
                idx_v[pl.ds(off + SPLIT + k * LANES, LANES)]
        for d in _gather_descs(b):
            d.start()

    _start(0, 0)
    _start(1, 1)

    lane = lax.broadcasted_iota(jnp.int32, (LANES,), 0)
    tidx = (lane // N_OUT) * LANES + (lane % N_OUT)
    inv = jnp.float32(1.0 / SEQ)

    def _process(g, b):
        buf = bufs[b]
        for d in _gather_descs(b):
            d.wait()
        for j in range(G):
            def rbody(r, acc, j=j):
                return acc + buf[j * SEQ + r, pl.ds(0, LANES)]
            acc = lax.fori_loop(0, SEQ, rbody, jnp.zeros((LANES,), jnp.float32))
            tsc_v[pl.ds(j * LANES, LANES)] = acc

        @pl.when(g + 2 < NCH)
        def _():
            _start(g + 2, b)

        vout = _lgather(tsc_v, tidx) * inv + bias_v[...]
        outst_v[pl.ds(g * LANES, LANES)] = vout

    def lbody(i, carry):
        _process(2 * i, 0)
        _process(2 * i + 1, 1)
        return carry

    lax.fori_loop(0, NCH // 2, lbody, 0)

    pltpu.sync_copy(
        outst_v,
        out_hbm.at[pl.ds(pl.multiple_of(wid * (NCH * LANES), 8), NCH * LANES)])


_pool = pl.kernel(
    _pool_body,
    out_type=jax.ShapeDtypeStruct((B * N_OUT,), jnp.float32),
    mesh=plsc.VectorSubcoreMesh(core_axis_name="c", subcore_axis_name="s"),
    compiler_params=pltpu.CompilerParams(
        needs_layout_passes=False, use_tc_tiling_on_sc=True),
    scratch_types=[
        pltpu.VMEM((IDXW,), jnp.int32),
        pltpu.VMEM((GIDX, PD), jnp.float32),
        pltpu.VMEM((GIDX, PD), jnp.float32),
        pltpu.VMEM((LANES,), jnp.float32),
        pltpu.VMEM((NCH * LANES,), jnp.float32),
        pltpu.VMEM((G * LANES,), jnp.float32),
        pltpu.VMEM((SPLIT,), jnp.int32),
        pltpu.VMEM((GIDX - SPLIT,), jnp.int32),
        pltpu.VMEM((SPLIT,), jnp.int32),
        pltpu.VMEM((GIDX - SPLIT,), jnp.int32),
        pltpu.SemaphoreType.DMA,
        pltpu.SemaphoreType.DMA,
    ],
)


def kernel(x, weights, fc_w, fc_b):
    fcw128 = jnp.zeros((PD, DW), fc_w.dtype).at[:N_OUT].set(fc_w)
    p = _proj(weights, fcw128)
    x_flat = x.reshape(-1)
    bias16 = jnp.tile(fc_b, LANES // N_OUT)
    out_flat = _pool(x_flat, p, bias16)
    return out_flat.reshape(B, N_OUT)


# speedup vs baseline: 3.4187x; 3.4187x over previous
import jax
import jax.numpy as jnp
from jax import lax
from jax.experimental import pallas as pl
from jax.experimental.pallas import tpu as pltpu
from jax.experimental.pallas import tpu_sc as plsc

B = 4096
SEQ = 50
DW = 300
VR = 1000001
N_OUT = 4
LANES = 16
PD = 128
NW = 32
RPW = B // NW
G = 4
NCH = RPW // G
GIDX = G * SEQ
SPLIT = 104
IDXW = RPW * SEQ
BM = 8192


def _lgather(ref, idx):
    return plsc.load_gather(ref, [idx])


def _proj_body(wt_ref, f_ref, out_ref):
    out_ref[...] = lax.dot_general(
        wt_ref[...].astype(jnp.bfloat16), f_ref[...].astype(jnp.bfloat16),
        (((0,), (0,)), ((), ())),
        preferred_element_type=jnp.float32)


_proj = pl.pallas_call(
    _proj_body,
    grid=(pl.cdiv(VR, BM),),
    in_specs=[
        pl.BlockSpec((DW, BM), lambda i: (0, i)),
        pl.BlockSpec((DW, PD), lambda i: (0, 0)),
    ],
    out_specs=pl.BlockSpec((BM, PD), lambda i: (i, 0)),
    out_shape=jax.ShapeDtypeStruct((VR, PD), jnp.float32),
)


def _pool_body(x_hbm, p_hbm, bias_hbm, out_hbm,
               idx_v, buf0, buf1, bias_v, outst_v, tsc_v,
               idxa0, idxb0, idxa1, idxb1, sem0, sem1):
    cid = lax.axis_index("c")
    sid = lax.axis_index("s")
    wid = sid * 2 + cid

    pltpu.sync_copy(x_hbm.at[pl.ds(pl.multiple_of(wid * IDXW, 8), IDXW)], idx_v)
    pltpu.sync_copy(bias_hbm, bias_v)

    bufs = (buf0, buf1)
    sems = (sem0, sem1)
    idxas = (idxa0, idxa1)
    idxbs = (idxb0, idxb1)

    def _gather_descs(b):
        d0 = pltpu.make_async_copy(
            p_hbm.at[idxas[b]], bufs[b].at[pl.ds(0, SPLIT)], sems[b])
        d1 = pltpu.make_async_copy(
            p_hbm.at[idxbs[b]], bufs[b].at[pl.ds(SPLIT, GIDX - SPLIT)],
            sems[b])
        return d0, d1

    def _start(g, b):
        off = g * GIDX
        for k in range(SPLIT // LANES):
            idxas[b][pl.ds(k * LANES, LANES)] = \
                idx_v[pl.ds(off + k * LANES, LANES)]
        idxas[b][pl.ds(SPLIT - LANES, LANES)] = \
            idx_v[pl.ds(off + SPLIT - LANES, LANES)]
        for k in range((GIDX - SPLIT) // LANES):
            idxbs[b][pl.ds(k * LANES, LANES)] = \
                idx_v[pl.ds(off + SPLIT + k * LANES, LANES)]
        for d in _gather_descs(b):
            d.start()

    _start(0, 0)
    _start(1, 1)

    lane = lax.broadcasted_iota(jnp.int32, (LANES,), 0)
    tidx = (lane // N_OUT) * LANES + (lane % N_OUT)
    inv = jnp.float32(1.0 / SEQ)

    def _process(g, b):
        buf = bufs[b]
        for d in _gather_descs(b):
            d.wait()
        for j in range(G):
            def rbody(r, acc, j=j):
                return acc + buf[j * SEQ + r, pl.ds(0, LANES)]
            acc = lax.fori_loop(0, SEQ, rbody, jnp.zeros((LANES,), jnp.float32))
            tsc_v[pl.ds(j * LANES, LANES)] = acc

        @pl.when(g + 2 < NCH)
        def _():
            _start(g + 2, b)

        vout = _lgather(tsc_v, tidx) * inv + bias_v[...]
        outst_v[pl.ds(g * LANES, LANES)] = vout

    def lbody(i, carry):
        _process(2 * i, 0)
        _process(2 * i + 1, 1)
        return carry

    lax.fori_loop(0, NCH // 2, lbody, 0)

    pltpu.sync_copy(
        outst_v,
        out_hbm.at[pl.ds(pl.multiple_of(wid * (NCH * LANES), 8), NCH * LANES)])


_pool = pl.kernel(
    _pool_body,
    out_type=jax.ShapeDtypeStruct((B * N_OUT,), jnp.float32),
    mesh=plsc.VectorSubcoreMesh(core_axis_name="c", subcore_axis_name="s"),
    compiler_params=pltpu.CompilerParams(
        needs_layout_passes=False, use_tc_tiling_on_sc=True),
    scratch_types=[
        pltpu.VMEM((IDXW,), jnp.int32),
        pltpu.VMEM((GIDX, PD), jnp.float32),
        pltpu.VMEM((GIDX, PD), jnp.float32),
        pltpu.VMEM((LANES,), jnp.float32),
        pltpu.VMEM((NCH * LANES,), jnp.float32),
        pltpu.VMEM((G * LANES,), jnp.float32),
        pltpu.VMEM((SPLIT,), jnp.int32),
        pltpu.VMEM((GIDX - SPLIT,), jnp.int32),
        pltpu.VMEM((SPLIT,), jnp.int32),
        pltpu.VMEM((GIDX - SPLIT,), jnp.int32),
        pltpu.SemaphoreType.DMA,
        pltpu.SemaphoreType.DMA,
    ],
)


def kernel(x, weights, fc_w, fc_b):
    fcw_t = jnp.zeros((DW, PD), fc_w.dtype).at[:, :N_OUT].set(fc_w.T)
    p = _proj(weights.T, fcw_t)
    x_flat = x.reshape(-1)
    bias16 = jnp.tile(fc_b, LANES // N_OUT)
    out_flat = _pool(x_flat, p, bias16)
    return out_flat.reshape(B, N_OUT)


# speedup vs baseline: 4.1409x; 1.2112x over previous
import jax
import jax.numpy as jnp
from jax import lax
from jax.experimental import pallas as pl
from jax.experimental.pallas import tpu as pltpu
from jax.experimental.pallas import tpu_sc as plsc

B = 4096
SEQ = 50
DW = 300
VR = 1000001
N_OUT = 4
LANES = 16
PD = 128
NPACK = 16
S = 62592
NW = 32
RPW = B // NW
G = 4
NCH = RPW // G
GIDX = G * SEQ
SPLIT = 104
IDXW = RPW * SEQ
BM2 = 384
GSTEPS = S // BM2


def _lgather(ref, idx):
    return plsc.load_gather(ref, [idx])


def _lgather2(ref, ridx, cidx):
    return plsc.load_gather(ref, [ridx, cidx])


def _proj_body(*refs):
    wt_refs = refs[:NPACK]
    f_ref = refs[NPACK]
    out_ref = refs[NPACK + 1]
    acc = None
    for j in range(NPACK):
        d = lax.dot_general(
            wt_refs[j][...].astype(jnp.bfloat16),
            f_ref[:, pl.ds(j * PD, PD)].astype(jnp.bfloat16),
            (((0,), (0,)), ((), ())),
            preferred_element_type=jnp.float32)
        acc = d if acc is None else acc + d
    out_ref[...] = acc


VLB = pl.cdiv(VR, BM2)


def _make_wt_spec(j):
    return pl.BlockSpec(
        (DW, BM2), lambda i, j=j: (0, jnp.minimum(j * GSTEPS + i, VLB - 1)))


_proj = pl.pallas_call(
    _proj_body,
    grid=(GSTEPS,),
    in_specs=[_make_wt_spec(j) for j in range(NPACK)]
    + [pl.BlockSpec((DW, NPACK * PD), lambda i: (0, 0))],
    out_specs=pl.BlockSpec((BM2, PD), lambda i: (i, 0)),
    out_shape=jax.ShapeDtypeStruct((S, PD), jnp.float32),
)


def _pool_body(x_hbm, off_hbm, p_hbm, bias_hbm, out_hbm,
               idx_v, offs_v, buf0, buf1, bias_v, outst_v, tsc_v,
               idxa0, idxb0, idxa1, idxb1, sem0, sem1):
    cid = lax.axis_index("c")
    sid = lax.axis_index("s")
    wid = sid * 2 + cid

    base = pl.multiple_of(wid * IDXW, 8)
    pltpu.sync_copy(x_hbm.at[pl.ds(base, IDXW)], idx_v)
    pltpu.sync_copy(off_hbm.at[pl.ds(base, IDXW)], offs_v)
    pltpu.sync_copy(bias_hbm, bias_v)

    bufs = (buf0, buf1)
    sems = (sem0, sem1)
    idxas = (idxa0, idxa1)
    idxbs = (idxb0, idxb1)

    def _gather_descs(b):
        d0 = pltpu.make_async_copy(
            p_hbm.at[idxas[b]], bufs[b].at[pl.ds(0, SPLIT)], sems[b])
        d1 = pltpu.make_async_copy(
            p_hbm.at[idxbs[b]], bufs[b].at[pl.ds(SPLIT, GIDX - SPLIT)],
            sems[b])
        return d0, d1

    def _start(g, b):
        off = g * GIDX
        for k in range(SPLIT // LANES):
            idxas[b][pl.ds(k * LANES, LANES)] = \
                idx_v[pl.ds(off + k * LANES, LANES)]
        idxas[b][pl.ds(SPLIT - LANES, LANES)] = \
            idx_v[pl.ds(off + SPLIT - LANES, LANES)]
        for k in range((GIDX - SPLIT) // LANES):
            idxbs[b][pl.ds(k * LANES, LANES)] = \
                idx_v[pl.ds(off + SPLIT + k * LANES, LANES)]
        for d in _gather_descs(b):
            d.start()

    _start(0, 0)
    _start(1, 1)

    lane = lax.broadcasted_iota(jnp.int32, (LANES,), 0)
    pairsel = lane // 8
    lane7 = lane % 8
    tidx = (lane // N_OUT) * LANES + (lane % N_OUT)
    inv = jnp.float32(1.0 / SEQ)

    def _process(g, b):
        buf = bufs[b]
        for d in _gather_descs(b):
            d.wait()
        goff = g * GIDX
        for j in range(G):
            def pbody(p, acc, j=j):
                row = j * SEQ + 2 * p + pairsel
                colv = _lgather(offs_v, goff + row) + lane7
                return acc + _lgather2(buf, row, colv)
            acc = lax.fori_loop(0, SEQ // 2, pbody,
                                jnp.zeros((LANES,), jnp.float32))
            tsc_v[pl.ds(j * LANES, LANES)] = acc

        @pl.when(g + 2 < NCH)
        def _():
            _start(g + 2, b)

        vout = (_lgather(tsc_v, tidx) + _lgather(tsc_v, tidx + 8)) * inv \
            + bias_v[...]
        outst_v[pl.ds(g * LANES, LANES)] = vout

    def lbody(i, carry):
        _process(2 * i, 0)
        _process(2 * i + 1, 1)
        return carry

    lax.fori_loop(0, NCH // 2, lbody, 0)

    pltpu.sync_copy(
        outst_v,
        out_hbm.at[pl.ds(pl.multiple_of(wid * (NCH * LANES), 8), NCH * LANES)])


_pool = pl.kernel(
    _pool_body,
    out_type=jax.ShapeDtypeStruct((B * N_OUT,), jnp.float32),
    mesh=plsc.VectorSubcoreMesh(core_axis_name="c", subcore_axis_name="s"),
    compiler_params=pltpu.CompilerParams(
        needs_layout_passes=False, use_tc_tiling_on_sc=True),
    scratch_types=[
        pltpu.VMEM((IDXW,), jnp.int32),
        pltpu.VMEM((IDXW,), jnp.int32),
        pltpu.VMEM((GIDX, PD), jnp.float32),
        pltpu.VMEM((GIDX, PD), jnp.float32),
        pltpu.VMEM((LANES,), jnp.float32),
        pltpu.VMEM((NCH * LANES,), jnp.float32),
        pltpu.VMEM((G * LANES,), jnp.float32),
        pltpu.VMEM((SPLIT,), jnp.int32),
        pltpu.VMEM((GIDX - SPLIT,), jnp.int32),
        pltpu.VMEM((SPLIT,), jnp.int32),
        pltpu.VMEM((GIDX - SPLIT,), jnp.int32),
        pltpu.SemaphoreType.DMA,
        pltpu.SemaphoreType.DMA,
    ],
)


def kernel(x, weights, fc_w, fc_b):
    fs = jnp.zeros((DW, NPACK, PD), fc_w.dtype)
    for j in range(NPACK):
        fs = fs.at[:, j, 8 * j:8 * j + N_OUT].set(fc_w.T)
    p = _proj(weights.T, *([weights.T] * (NPACK - 1)),
              fs.reshape(DW, NPACK * PD))
    x_flat = x.reshape(-1)
    xj = x_flat // S
    x_g = x_flat - xj * S
    x_off = xj * 8
    bias16 = jnp.tile(fc_b, LANES // N_OUT)
    out_flat = _pool(x_g, x_off, p, bias16)
    return out_flat.reshape(B, N_OUT)
